# baseline (speedup 1.0000x reference)
import math

import jax
import jax.numpy as jnp
from jax.experimental import pallas as pl
from jax.experimental.pallas import tpu as pltpu


def _pool_kernel(x_ref, out_ref):
    m = x_ref.shape[1]
    out_ref[...] = jnp.sum(x_ref[...], axis=1, keepdims=True) * (1.0 / m)


def _score_kernel(xp_ref, pk_ref, hist_ref, prompt_ref, w1_ref, b1_ref,
                  g_ref, bb_ref, w2_ref, b2_ref,
                  fs_ref, sel_ref, div_ref, pr_ref, prn_ref):
    xp = xp_ref[...]
    pk = pk_ref[...]
    xn = xp / jnp.maximum(jnp.sqrt(jnp.sum(xp * xp, axis=1, keepdims=True)), 1e-12)
    pkn = pk / jnp.maximum(jnp.sqrt(jnp.sum(pk * pk, axis=1, keepdims=True)), 1e-12)
    sim = jnp.dot(xn, pkn.T, preferred_element_type=jnp.float32)

    hist = hist_ref[...]
    usage = jnp.sum(hist, axis=0, keepdims=True) * (1.0 / hist.shape[0])
    div = 1.0 - usage

    feats = pk[None, :, :] * xp[:, None, :]
    b_, p_, d_ = feats.shape
    feats2 = feats.reshape(b_ * p_, d_)
    h = jnp.dot(feats2, w1_ref[...], preferred_element_type=jnp.float32) + b1_ref[...]
    mu = jnp.mean(h, axis=-1, keepdims=True)
    var = jnp.mean((h - mu) ** 2, axis=-1, keepdims=True)
    h = (h - mu) / jnp.sqrt(var + 1e-5) * g_ref[...] + bb_ref[...]
    h = 0.5 * h * (1.0 + jax.lax.erf(h * (1.0 / math.sqrt(2.0))))
    rel = (jnp.dot(h, w2_ref[...], preferred_element_type=jnp.float32)
           + b2_ref[...]).reshape(b_, p_)
    rel = rel - jnp.max(rel, axis=1, keepdims=True)
    e = jnp.exp(rel)
    relevance = e / jnp.sum(e, axis=1, keepdims=True)

    fs = 0.5 * sim + 0.3 * div + 0.2 * relevance
    fs_ref[...] = fs
    batch_scores = jnp.mean(fs, axis=0)
    sel = jnp.argmax(batch_scores).astype(jnp.int32)
    sel_ref[...] = sel.reshape(1, 1)
    div_ref[...] = div

    pr = prompt_ref[sel]
    pr_ref[...] = pr
    prn_ref[...] = pr / jnp.maximum(
        jnp.sqrt(jnp.sum(pr * pr, axis=1, keepdims=True)), 1e-12)


def _attend_kernel(dp_ref, pr_ref, prn_ref, out_ref):
    dp = dp_ref[...]
    inv_sqrt_d = 1.0 / math.sqrt(dp.shape[1])
    rn2 = jnp.sum(dp * dp, axis=1, keepdims=True)
    scale = jax.lax.rsqrt(jnp.maximum(rn2, 1e-24)) * inv_sqrt_d
    attn = jnp.dot(dp, prn_ref[...].T, preferred_element_type=jnp.float32)
    e = jnp.exp(attn * scale)
    w = e / jnp.sum(e, axis=1, keepdims=True)
    out_ref[...] = jnp.dot(w, pr_ref[...], preferred_element_type=jnp.float32)


def kernel(x_embed, depth_feature, prompt, prompt_key, history_buffer,
           W1, b1, ln_g, ln_b, W2, b2):
    B, M, D = x_embed.shape
    P, L, _ = prompt.shape
    TOP_K = 1

    BB = 2
    x_pooled = pl.pallas_call(
        _pool_kernel,
        grid=(B // BB,),
        in_specs=[pl.BlockSpec((BB, M, D), lambda b: (b, 0, 0))],
        out_specs=pl.BlockSpec((BB, 1, D), lambda b: (b, 0, 0)),
        out_shape=jax.ShapeDtypeStruct((B, 1, D), jnp.float32),
        compiler_params=pltpu.CompilerParams(
            dimension_semantics=("arbitrary",)),
    )(x_embed)
    x_pooled = x_pooled.reshape(B, D)

    fs, sel, div_row, pr_sel, prn_sel = pl.pallas_call(
        _score_kernel,
        out_shape=(
            jax.ShapeDtypeStruct((B, P), jnp.float32),
            jax.ShapeDtypeStruct((1, 1), jnp.int32),
            jax.ShapeDtypeStruct((1, P), jnp.float32),
            jax.ShapeDtypeStruct((L, D), jnp.float32),
            jax.ShapeDtypeStruct((L, D), jnp.float32),
        ),
    )(x_pooled, prompt_key, history_buffer, prompt,
      W1, b1.reshape(1, -1), ln_g.reshape(1, -1), ln_b.reshape(1, -1),
      W2, b2.reshape(1, 1))

    N = B * M
    MB = 8192
    dp2 = depth_feature.reshape(N, D)
    prompted = pl.pallas_call(
        _attend_kernel,
        grid=(N // MB,),
        in_specs=[
            pl.BlockSpec((MB, D), lambda i: (i, 0)),
            pl.BlockSpec((L, D), lambda i: (0, 0)),
            pl.BlockSpec((L, D), lambda i: (0, 0)),
        ],
        out_specs=pl.BlockSpec((MB, D), lambda i: (i, 0)),
        out_shape=jax.ShapeDtypeStruct((N, D), jnp.float32),
        compiler_params=pltpu.CompilerParams(
            dimension_semantics=("arbitrary",)),
    )(dp2, pr_sel, prn_sel)
    prompted = prompted.reshape(B, M, D)

    selected_idx = jnp.broadcast_to(sel.reshape(1, 1), (B, TOP_K))
    return (prompted, fs, selected_idx, div_row.reshape(P))


# speedup vs baseline: 1.0266x; 1.0266x over previous
import math

import jax
import jax.numpy as jnp
from jax.experimental import pallas as pl
from jax.experimental.pallas import tpu as pltpu

_BB = 2
_MB = 8192


def _mega_kernel(x_ref, dp_ref, pk_ref, hist_ref, prompt_ref, w1_ref, b1_ref,
                 g_ref, bb_ref, w2_ref, b2_ref,
                 out_ref, fs_ref, sel_ref, div_ref,
                 xp_scr, pr_scr, prn_scr):
    i = pl.program_id(0)
    npool = xp_scr.shape[0]

    @pl.when(i < npool)
    def _pool():
        m = x_ref.shape[1]
        xp_scr[i] = jnp.sum(x_ref[...], axis=1) * (1.0 / m)

    @pl.when(i == npool)
    def _score():
        xp = xp_scr[...].reshape(-1, xp_scr.shape[2])
        pk = pk_ref[...]
        xn = xp / jnp.maximum(jnp.sqrt(jnp.sum(xp * xp, axis=1, keepdims=True)), 1e-12)
        pkn = pk / jnp.maximum(jnp.sqrt(jnp.sum(pk * pk, axis=1, keepdims=True)), 1e-12)
        sim = jnp.dot(xn, pkn.T, preferred_element_type=jnp.float32)

        hist = hist_ref[...]
        usage = jnp.sum(hist, axis=0, keepdims=True) * (1.0 / hist.shape[0])
        div = 1.0 - usage

        feats = pk[None, :, :] * xp[:, None, :]
        b_, p_, d_ = feats.shape
        feats2 = feats.reshape(b_ * p_, d_)
        h = jnp.dot(feats2, w1_ref[...], preferred_element_type=jnp.float32) + b1_ref[...]
        mu = jnp.mean(h, axis=-1, keepdims=True)
        var = jnp.mean((h - mu) ** 2, axis=-1, keepdims=True)
        h = (h - mu) / jnp.sqrt(var + 1e-5) * g_ref[...] + bb_ref[...]
        h = 0.5 * h * (1.0 + jax.lax.erf(h * (1.0 / math.sqrt(2.0))))
        rel = (jnp.dot(h, w2_ref[...], preferred_element_type=jnp.float32)
               + b2_ref[...]).reshape(b_, p_)
        rel = rel - jnp.max(rel, axis=1, keepdims=True)
        e = jnp.exp(rel)
        relevance = e / jnp.sum(e, axis=1, keepdims=True)

        fs = 0.5 * sim + 0.3 * div + 0.2 * relevance
        fs_ref[...] = fs
        sel = jnp.argmax(jnp.mean(fs, axis=0)).astype(jnp.int32)
        sel_ref[...] = sel.reshape(1, 1)
        div_ref[...] = div

        pr = prompt_ref[sel]
        pr_scr[...] = pr
        prn_scr[...] = pr / jnp.maximum(
            jnp.sqrt(jnp.sum(pr * pr, axis=1, keepdims=True)), 1e-12)

    @pl.when(i >= npool)
    def _attend():
        dp = dp_ref[...]
        inv_sqrt_d = 1.0 / math.sqrt(dp.shape[1])
        rn2 = jnp.sum(dp * dp, axis=1, keepdims=True)
        scale = jax.lax.rsqrt(jnp.maximum(rn2, 1e-24)) * inv_sqrt_d
        attn = jnp.dot(dp, prn_scr[...].T, preferred_element_type=jnp.float32)
        e = jnp.exp(attn * scale)
        w = e / jnp.sum(e, axis=1, keepdims=True)
        out_ref[...] = jnp.dot(w, pr_scr[...], preferred_element_type=jnp.float32)


def kernel(x_embed, depth_feature, prompt, prompt_key, history_buffer,
           W1, b1, ln_g, ln_b, W2, b2):
    B, M, D = x_embed.shape
    P, L, _ = prompt.shape
    TOP_K = 1
    N = B * M
    npool = B // _BB
    natt = N // _MB
    dp2 = depth_feature.reshape(N, D)

    prompted, fs, sel, div_row = pl.pallas_call(
        _mega_kernel,
        grid=(npool + natt,),
        in_specs=[
            pl.BlockSpec((_BB, M, D), lambda i: (jnp.minimum(i, npool - 1), 0, 0)),
            pl.BlockSpec((_MB, D), lambda i: (jnp.maximum(i - npool, 0), 0)),
            pl.BlockSpec((P, D), lambda i: (0, 0)),
            pl.BlockSpec(history_buffer.shape, lambda i: (0, 0)),
            pl.BlockSpec((P, L, D), lambda i: (0, 0, 0)),
            pl.BlockSpec(W1.shape, lambda i: (0, 0)),
            pl.BlockSpec((1, W1.shape[1]), lambda i: (0, 0)),
            pl.BlockSpec((1, W1.shape[1]), lambda i: (0, 0)),
            pl.BlockSpec((1, W1.shape[1]), lambda i: (0, 0)),
            pl.BlockSpec(W2.shape, lambda i: (0, 0)),
            pl.BlockSpec((1, 1), lambda i: (0, 0)),
        ],
        out_specs=(
            pl.BlockSpec((_MB, D), lambda i: (jnp.maximum(i - npool, 0), 0)),
            pl.BlockSpec((B, P), lambda i: (0, 0)),
            pl.BlockSpec((1, 1), lambda i: (0, 0)),
            pl.BlockSpec((1, P), lambda i: (0, 0)),
        ),
        out_shape=(
            jax.ShapeDtypeStruct((N, D), jnp.float32),
            jax.ShapeDtypeStruct((B, P), jnp.float32),
            jax.ShapeDtypeStruct((1, 1), jnp.int32),
            jax.ShapeDtypeStruct((1, P), jnp.float32),
        ),
        scratch_shapes=[
            pltpu.VMEM((npool, _BB, D), jnp.float32),
            pltpu.VMEM((L, D), jnp.float32),
            pltpu.VMEM((L, D), jnp.float32),
        ],
        compiler_params=pltpu.CompilerParams(
            dimension_semantics=("arbitrary",)),
    )(x_embed, dp2, prompt_key, history_buffer, prompt,
      W1, b1.reshape(1, -1), ln_g.reshape(1, -1), ln_b.reshape(1, -1),
      W2, b2.reshape(1, 1))

    prompted = prompted.reshape(B, M, D)
    selected_idx = jnp.broadcast_to(sel.reshape(1, 1), (B, TOP_K))
    return (prompted, fs, selected_idx, div_row.reshape(P))


# speedup vs baseline: 1.0617x; 1.0341x over previous
import math

import jax
import jax.numpy as jnp
from jax.experimental import pallas as pl
from jax.experimental.pallas import tpu as pltpu

_BB = 4
_MB = 8192


def _mega_kernel(x_ref, dp_ref, pk_ref, hist_ref, prompt_ref, w1_ref, b1_ref,
                 g_ref, bb_ref, w2_ref, b2_ref,
                 out_ref, fs_ref, sel_ref, div_ref,
                 xp_scr, pr_scr, prn_scr):
    i = pl.program_id(0)
    npool = xp_scr.shape[0]

    @pl.when(i < npool)
    def _pool():
        m = x_ref.shape[1]
        xp_scr[i] = jnp.sum(x_ref[...], axis=1) * (1.0 / m)

    @pl.when(i == npool)
    def _score():
        xp = xp_scr[...].reshape(-1, xp_scr.shape[2])
        pk = pk_ref[...]
        xn = xp / jnp.maximum(jnp.sqrt(jnp.sum(xp * xp, axis=1, keepdims=True)), 1e-12)
        pkn = pk / jnp.maximum(jnp.sqrt(jnp.sum(pk * pk, axis=1, keepdims=True)), 1e-12)
        sim = jnp.dot(xn, pkn.T, preferred_element_type=jnp.float32)

        hist = hist_ref[...]
        usage = jnp.sum(hist, axis=0, keepdims=True) * (1.0 / hist.shape[0])
        div = 1.0 - usage

        feats = pk[None, :, :] * xp[:, None, :]
        b_, p_, d_ = feats.shape
        feats2 = feats.reshape(b_ * p_, d_)
        h = jnp.dot(feats2, w1_ref[...], preferred_element_type=jnp.float32) + b1_ref[...]
        mu = jnp.mean(h, axis=-1, keepdims=True)
        var = jnp.mean((h - mu) ** 2, axis=-1, keepdims=True)
        h = (h - mu) / jnp.sqrt(var + 1e-5) * g_ref[...] + bb_ref[...]
        h = 0.5 * h * (1.0 + jax.lax.erf(h * (1.0 / math.sqrt(2.0))))
        rel = (jnp.dot(h, w2_ref[...], preferred_element_type=jnp.float32)
               + b2_ref[...]).reshape(b_, p_)
        rel = rel - jnp.max(rel, axis=1, keepdims=True)
        e = jnp.exp(rel)
        relevance = e / jnp.sum(e, axis=1, keepdims=True)

        fs = 0.5 * sim + 0.3 * div + 0.2 * relevance
        fs_ref[...] = fs
        sel = jnp.argmax(jnp.mean(fs, axis=0)).astype(jnp.int32)
        sel_ref[...] = sel.reshape(1, 1)
        div_ref[...] = div

        pr = prompt_ref[sel]
        pr_scr[...] = pr
        prn_scr[...] = pr / jnp.maximum(
            jnp.sqrt(jnp.sum(pr * pr, axis=1, keepdims=True)), 1e-12)

    @pl.when(i >= npool)
    def _attend():
        dp = dp_ref[...]
        inv_sqrt_d = 1.0 / math.sqrt(dp.shape[1])
        rn2 = jnp.sum(dp * dp, axis=1, keepdims=True)
        scale = jax.lax.rsqrt(jnp.maximum(rn2, 1e-24)) * inv_sqrt_d
        attn = jnp.dot(dp, prn_scr[...].T, preferred_element_type=jnp.float32)
        e = jnp.exp(attn * scale)
        w = e / jnp.sum(e, axis=1, keepdims=True)
        out_ref[...] = jnp.dot(w, pr_scr[...], preferred_element_type=jnp.float32)


def kernel(x_embed, depth_feature, prompt, prompt_key, history_buffer,
           W1, b1, ln_g, ln_b, W2, b2):
    B, M, D = x_embed.shape
    P, L, _ = prompt.shape
    TOP_K = 1
    N = B * M
    npool = B // _BB
    natt = N // _MB
    dp2 = depth_feature.reshape(N, D)

    prompted, fs, sel, div_row = pl.pallas_call(
        _mega_kernel,
        grid=(npool + natt,),
        in_specs=[
            pl.BlockSpec((_BB, M, D), lambda i: (jnp.minimum(i, npool - 1), 0, 0)),
            pl.BlockSpec((_MB, D), lambda i: (jnp.maximum(i - npool, 0), 0)),
            pl.BlockSpec((P, D), lambda i: (0, 0)),
            pl.BlockSpec(history_buffer.shape, lambda i: (0, 0)),
            pl.BlockSpec((P, L, D), lambda i: (0, 0, 0)),
            pl.BlockSpec(W1.shape, lambda i: (0, 0)),
            pl.BlockSpec((1, W1.shape[1]), lambda i: (0, 0)),
            pl.BlockSpec((1, W1.shape[1]), lambda i: (0, 0)),
            pl.BlockSpec((1, W1.shape[1]), lambda i: (0, 0)),
            pl.BlockSpec(W2.shape, lambda i: (0, 0)),
            pl.BlockSpec((1, 1), lambda i: (0, 0)),
        ],
        out_specs=(
            pl.BlockSpec((_MB, D), lambda i: (jnp.maximum(i - npool, 0), 0)),
            pl.BlockSpec((B, P), lambda i: (0, 0)),
            pl.BlockSpec((1, 1), lambda i: (0, 0)),
            pl.BlockSpec((1, P), lambda i: (0, 0)),
        ),
        out_shape=(
            jax.ShapeDtypeStruct((N, D), jnp.float32),
            jax.ShapeDtypeStruct((B, P), jnp.float32),
            jax.ShapeDtypeStruct((1, 1), jnp.int32),
            jax.ShapeDtypeStruct((1, P), jnp.float32),
        ),
        scratch_shapes=[
            pltpu.VMEM((npool, _BB, D), jnp.float32),
            pltpu.VMEM((L, D), jnp.float32),
            pltpu.VMEM((L, D), jnp.float32),
        ],
        compiler_params=pltpu.CompilerParams(
            dimension_semantics=("arbitrary",)),
    )(x_embed, dp2, prompt_key, history_buffer, prompt,
      W1, b1.reshape(1, -1), ln_g.reshape(1, -1), ln_b.reshape(1, -1),
      W2, b2.reshape(1, 1))

    prompted = prompted.reshape(B, M, D)
    selected_idx = jnp.broadcast_to(sel.reshape(1, 1), (B, TOP_K))
    return (prompted, fs, selected_idx, div_row.reshape(P))
